# baseline (speedup 1.0000x reference)
import functools

import jax
import jax.numpy as jnp
from jax import lax
from jax.experimental import pallas as pl
from jax.experimental.pallas import tpu as pltpu
from jax.experimental.pallas import tpu_sc as plsc

VOCAB = 64001
DIM = 128
HALF = DIM // 2
MAX_LEN = 200
B = 4096
TOK = B * MAX_LEN
EPS = 1e-12


def _transform_body(w_ref, w2_ref, o_ref):
    t = lax.dot_general(
        w_ref[...], w2_ref[...], (((1,), (1,)), ((), ())),
        preferred_element_type=jnp.float32)
    o_ref[...] = t.astype(jnp.bfloat16)


def _transform_table(word_table, W2):
    R = 512
    return pl.pallas_call(
        _transform_body,
        grid=(pl.cdiv(VOCAB, R),),
        in_specs=[pl.BlockSpec((R, DIM), lambda i: (i, 0)),
                  pl.BlockSpec((DIM, DIM), lambda i: (0, 0))],
        out_specs=pl.BlockSpec((R, DIM), lambda i: (i, 0)),
        out_shape=jax.ShapeDtypeStruct((VOCAB, DIM), jnp.bfloat16),
    )(word_table, W2)


_NW = 32
_B_PER_W = TOK // _NW
_CH = 256
_IR = _CH // 128
_NCH = _B_PER_W // _CH


def _sc_gather(table, ids2d):
    mesh = plsc.VectorSubcoreMesh(core_axis_name="c", subcore_axis_name="s")

    @functools.partial(
        pl.kernel,
        out_type=jax.ShapeDtypeStruct((TOK, DIM), jnp.bfloat16),
        mesh=mesh,
        scratch_types=[
            pltpu.VMEM((2, _IR, 128), jnp.int32),
            pltpu.VMEM((2, _CH, DIM), jnp.bfloat16),
            pltpu.SemaphoreType.DMA,
            pltpu.SemaphoreType.DMA,
            pltpu.SemaphoreType.DMA,
            pltpu.SemaphoreType.DMA,
        ],
        compiler_params=pltpu.CompilerParams(use_tc_tiling_on_sc=False),
    )
    def k(t_hbm, ids_hbm, out_hbm, idx_v, rows_v, sg0, sg1, so0, so1):
        wid = lax.axis_index("s") * 2 + lax.axis_index("c")
        sg = (sg0, sg1)
        so = (so0, so1)

        def issue(g, b):
            irow = wid * (_B_PER_W // 128) + g * _IR
            pltpu.sync_copy(ids_hbm.at[pl.ds(irow, _IR)], idx_v.at[b])
            for j in range(_IR):
                pltpu.async_copy(t_hbm.at[idx_v.at[b].at[j]],
                                 rows_v.at[b].at[pl.ds(j * 128, 128)], sg[b])

        def flush(g, b):
            for j in range(_IR):
                pltpu.make_async_copy(t_hbm.at[idx_v.at[b].at[j]],
                                      rows_v.at[b].at[pl.ds(j * 128, 128)],
                                      sg[b]).wait()
            base = wid * _B_PER_W + g * _CH
            pltpu.async_copy(rows_v.at[b], out_hbm.at[pl.ds(base, _CH)], so[b])

        def wait_out(b):
            pltpu.make_async_copy(rows_v.at[b], out_hbm.at[pl.ds(0, _CH)],
                                  so[b]).wait()

        issue(0, 0)

        def body(i, carry):
            g0 = 2 * i

            @pl.when(i >= 1)
            def _():
                wait_out(1)
            issue(g0 + 1, 1)
            flush(g0, 0)
            wait_out(0)

            @pl.when(g0 + 2 < _NCH)
            def _():
                issue(g0 + 2, 0)
            flush(g0 + 1, 1)
            return carry

        lax.fori_loop(0, _NCH // 2, body, 0)
        wait_out(1)

    return k(table, ids2d)


_BR = 16
_N = _BR * MAX_LEN


def _ln_body(g_ref, seg_ref, pos_ref, typ_ref, gam_ref, bet_ref, o_ref):
    x = g_ref[...].astype(jnp.float32)
    seg = seg_ref[...].reshape(_N, 1)
    oneh = (seg == lax.broadcasted_iota(jnp.int32, (_N, 8), 1)
            ).astype(jnp.float32)
    t = lax.dot_general(oneh, typ_ref[...], (((1,), (0,)), ((), ())),
                        preferred_element_type=jnp.float32)
    x = (x + pos_ref[...][None, :, :]).reshape(_N, DIM) + t
    ones = jnp.ones((DIM, 1), jnp.float32)
    dot = lambda a: lax.dot_general(a, ones, (((1,), (0,)), ((), ())),
                                    preferred_element_type=jnp.float32)
    ssum = dot(x)
    ssq = dot(x * x)
    mean = ssum * (1.0 / DIM)
    var = ssq * (1.0 / DIM) - mean * mean
    r = lax.rsqrt(var + EPS)
    y = (x - mean) * r * gam_ref[...] + bet_ref[...]
    o_ref[...] = y.reshape(_BR, MAX_LEN, DIM)


def _ln(gathered, segment_ids, pos_table, type_table, gamma, beta):
    full = lambda shape: pl.BlockSpec(shape, lambda i: tuple(0 for _ in shape))
    typ8 = jnp.zeros((8, DIM), jnp.float32).at[:3].set(type_table)
    return pl.pallas_call(
        _ln_body,
        grid=(B // _BR,),
        in_specs=[
            pl.BlockSpec((_BR, MAX_LEN, DIM), lambda i: (i, 0, 0)),
            pl.BlockSpec((_BR, MAX_LEN, 1), lambda i: (i, 0, 0)),
            full((MAX_LEN, DIM)),
            full((8, DIM)),
            full((1, DIM)), full((1, DIM)),
        ],
        out_specs=pl.BlockSpec((_BR, MAX_LEN, DIM), lambda i: (i, 0, 0)),
        out_shape=jax.ShapeDtypeStruct((B, MAX_LEN, DIM), jnp.float32),
    )(gathered, segment_ids.reshape(B, MAX_LEN, 1),
      pos_table, typ8,
      gamma.reshape(1, DIM), beta.reshape(1, DIM))


def kernel(input_ids, segment_ids, word_table, W2, pos_table, type_table,
           gamma, beta):
    table = _transform_table(word_table, W2)
    ids2d = input_ids.astype(jnp.int32).reshape(TOK // 128, 128)
    gathered = _sc_gather(table, ids2d)
    return _ln(gathered.reshape(B, MAX_LEN, DIM), segment_ids.astype(jnp.int32),
               pos_table, type_table, gamma, beta)


# speedup vs baseline: 1.0023x; 1.0023x over previous
import functools

import jax
import jax.numpy as jnp
from jax import lax
from jax.experimental import pallas as pl
from jax.experimental.pallas import tpu as pltpu
from jax.experimental.pallas import tpu_sc as plsc

VOCAB = 64001
DIM = 128
HALF = DIM // 2
MAX_LEN = 200
B = 4096
TOK = B * MAX_LEN
EPS = 1e-12


def _transform_body(w_ref, w2_ref, o_ref):
    t = lax.dot_general(
        w_ref[...], w2_ref[...], (((1,), (1,)), ((), ())),
        preferred_element_type=jnp.float32)
    o_ref[...] = t.astype(jnp.bfloat16)


def _transform_table(word_table, W2):
    R = 512
    return pl.pallas_call(
        _transform_body,
        grid=(pl.cdiv(VOCAB, R),),
        in_specs=[pl.BlockSpec((R, DIM), lambda i: (i, 0)),
                  pl.BlockSpec((DIM, DIM), lambda i: (0, 0))],
        out_specs=pl.BlockSpec((R, DIM), lambda i: (i, 0)),
        out_shape=jax.ShapeDtypeStruct((VOCAB, DIM), jnp.bfloat16),
    )(word_table, W2)


_NW = 32
_B_PER_W = TOK // _NW
_CH = 256
_IR = _CH // 128
_NCH = _B_PER_W // _CH


def _sc_gather(table, ids2d):
    mesh = plsc.VectorSubcoreMesh(core_axis_name="c", subcore_axis_name="s")

    @functools.partial(
        pl.kernel,
        out_type=jax.ShapeDtypeStruct((TOK, DIM), jnp.bfloat16),
        mesh=mesh,
        scratch_types=[
            pltpu.VMEM((2, _IR, 128), jnp.int32),
            pltpu.VMEM((2, _CH, DIM), jnp.bfloat16),
            pltpu.SemaphoreType.DMA,
            pltpu.SemaphoreType.DMA,
            pltpu.SemaphoreType.DMA,
            pltpu.SemaphoreType.DMA,
        ],
        compiler_params=pltpu.CompilerParams(use_tc_tiling_on_sc=False),
    )
    def k(t_hbm, ids_hbm, out_hbm, idx_v, rows_v, sg0, sg1, so0, so1):
        wid = lax.axis_index("s") * 2 + lax.axis_index("c")
        sg = (sg0, sg1)
        so = (so0, so1)

        def issue(g, b):
            irow = wid * (_B_PER_W // 128) + g * _IR
            pltpu.sync_copy(ids_hbm.at[pl.ds(irow, _IR)], idx_v.at[b])
            for j in range(_IR):
                pltpu.async_copy(t_hbm.at[idx_v.at[b].at[j]],
                                 rows_v.at[b].at[pl.ds(j * 128, 128)], sg[b])

        def flush(g, b):
            for j in range(_IR):
                pltpu.make_async_copy(t_hbm.at[idx_v.at[b].at[j]],
                                      rows_v.at[b].at[pl.ds(j * 128, 128)],
                                      sg[b]).wait()
            base = wid * _B_PER_W + g * _CH
            pltpu.async_copy(rows_v.at[b], out_hbm.at[pl.ds(base, _CH)], so[b])

        def wait_out(b):
            pltpu.make_async_copy(rows_v.at[b], out_hbm.at[pl.ds(0, _CH)],
                                  so[b]).wait()

        issue(0, 0)

        def body(i, carry):
            g0 = 2 * i

            @pl.when(i >= 1)
            def _():
                wait_out(1)
            issue(g0 + 1, 1)
            flush(g0, 0)
            wait_out(0)

            @pl.when(g0 + 2 < _NCH)
            def _():
                issue(g0 + 2, 0)
            flush(g0 + 1, 1)
            return carry

        lax.fori_loop(0, _NCH // 2, body, 0)
        wait_out(1)

    return k(table, ids2d)


_BR = 16
_N = _BR * MAX_LEN


def _ln_body(g_ref, seg_ref, pos_ref, typ_ref, gam_ref, bet_ref, o_ref):
    x = g_ref[...].astype(jnp.float32)
    seg = seg_ref[...]
    oneh = (seg == lax.broadcasted_iota(jnp.int32, (_N, 8), 1)
            ).astype(jnp.float32)
    t = lax.dot_general(oneh, typ_ref[...], (((1,), (0,)), ((), ())),
                        preferred_element_type=jnp.float32)
    x = x + pos_ref[...] + t
    ones = jnp.ones((DIM, 1), jnp.float32)
    dot = lambda a: lax.dot_general(a, ones, (((1,), (0,)), ((), ())),
                                    preferred_element_type=jnp.float32)
    ssum = dot(x)
    ssq = dot(x * x)
    mean = ssum * (1.0 / DIM)
    var = ssq * (1.0 / DIM) - mean * mean
    r = lax.rsqrt(var + EPS)
    o_ref[...] = (x - mean) * r * gam_ref[...] + bet_ref[...]


def _ln(gathered, segment_ids, pos16, type_table, gamma, beta):
    full = lambda shape: pl.BlockSpec(shape, lambda i: tuple(0 for _ in shape))
    typ8 = jnp.zeros((8, DIM), jnp.float32).at[:3].set(type_table)
    return pl.pallas_call(
        _ln_body,
        grid=(TOK // _N,),
        in_specs=[
            pl.BlockSpec((_N, DIM), lambda i: (i, 0)),
            pl.BlockSpec((_N, 1), lambda i: (i, 0)),
            full((_N, DIM)),
            full((8, DIM)),
            full((1, DIM)), full((1, DIM)),
        ],
        out_specs=pl.BlockSpec((_N, DIM), lambda i: (i, 0)),
        out_shape=jax.ShapeDtypeStruct((TOK, DIM), jnp.float32),
    )(gathered, segment_ids.reshape(TOK, 1),
      pos16, typ8,
      gamma.reshape(1, DIM), beta.reshape(1, DIM))


def kernel(input_ids, segment_ids, word_table, W2, pos_table, type_table,
           gamma, beta):
    table = _transform_table(word_table, W2)
    ids2d = input_ids.astype(jnp.int32).reshape(TOK // 128, 128)
    gathered = _sc_gather(table, ids2d)
    pos16 = jnp.tile(pos_table, (_BR, 1))
    out = _ln(gathered, segment_ids.astype(jnp.int32),
              pos16, type_table, gamma, beta)
    return out.reshape(B, MAX_LEN, DIM)


# speedup vs baseline: 1.6508x; 1.6470x over previous
import functools

import jax
import jax.numpy as jnp
from jax import lax
from jax.experimental import pallas as pl
from jax.experimental.pallas import tpu as pltpu
from jax.experimental.pallas import tpu_sc as plsc

VOCAB = 64001
DIM = 128
HALF = DIM // 2
MAX_LEN = 200
B = 4096
TOK = B * MAX_LEN
EPS = 1e-12


def _transform_body(w_ref, w2_ref, o_ref):
    o_ref[...] = lax.dot_general(
        w_ref[...], w2_ref[...], (((1,), (1,)), ((), ())),
        preferred_element_type=jnp.float32)


def _transform_table(word_table, W2):
    R = 512
    return pl.pallas_call(
        _transform_body,
        grid=(pl.cdiv(VOCAB, R),),
        in_specs=[pl.BlockSpec((R, DIM), lambda i: (i, 0)),
                  pl.BlockSpec((DIM, DIM), lambda i: (0, 0))],
        out_specs=pl.BlockSpec((R, DIM), lambda i: (i, 0)),
        out_shape=jax.ShapeDtypeStruct((VOCAB, DIM), jnp.float32),
    )(word_table, W2)


_NW = 32
_B_PER_W = TOK // _NW
_CH = 256
_IR = _CH // 128
_NCH = _B_PER_W // _CH


def _sc_gather(table, ids2d):
    mesh = plsc.VectorSubcoreMesh(core_axis_name="c", subcore_axis_name="s")

    @functools.partial(
        pl.kernel,
        out_type=jax.ShapeDtypeStruct((TOK, DIM), jnp.float32),
        mesh=mesh,
        scratch_types=[
            pltpu.VMEM((2, _IR, 128), jnp.int32),
            pltpu.VMEM((2, _CH, DIM), jnp.float32),
            pltpu.SemaphoreType.DMA,
            pltpu.SemaphoreType.DMA,
            pltpu.SemaphoreType.DMA,
            pltpu.SemaphoreType.DMA,
        ],
        compiler_params=pltpu.CompilerParams(use_tc_tiling_on_sc=False),
    )
    def k(t_hbm, ids_hbm, out_hbm, idx_v, rows_v, sg0, sg1, so0, so1):
        wid = lax.axis_index("s") * 2 + lax.axis_index("c")
        sg = (sg0, sg1)
        so = (so0, so1)

        def issue(g, b):
            irow = wid * (_B_PER_W // 128) + g * _IR
            pltpu.sync_copy(ids_hbm.at[pl.ds(irow, _IR)], idx_v.at[b])
            for j in range(_IR):
                pltpu.async_copy(t_hbm.at[idx_v.at[b].at[j]],
                                 rows_v.at[b].at[pl.ds(j * 128, 128)], sg[b])

        def flush(g, b):
            for j in range(_IR):
                pltpu.make_async_copy(t_hbm.at[idx_v.at[b].at[j]],
                                      rows_v.at[b].at[pl.ds(j * 128, 128)],
                                      sg[b]).wait()
            base = wid * _B_PER_W + g * _CH
            pltpu.async_copy(rows_v.at[b], out_hbm.at[pl.ds(base, _CH)], so[b])

        def wait_out(b):
            pltpu.make_async_copy(rows_v.at[b], out_hbm.at[pl.ds(0, _CH)],
                                  so[b]).wait()

        issue(0, 0)

        def body(i, carry):
            g0 = 2 * i

            @pl.when(i >= 1)
            def _():
                wait_out(1)
            issue(g0 + 1, 1)
            flush(g0, 0)
            wait_out(0)

            @pl.when(g0 + 2 < _NCH)
            def _():
                issue(g0 + 2, 0)
            flush(g0 + 1, 1)
            return carry

        lax.fori_loop(0, _NCH // 2, body, 0)
        wait_out(1)

    return k(table, ids2d)


_BR = 16
_N = _BR * MAX_LEN


def _ln_body(g_ref, seg_ref, pos_ref, typ_ref, gam_ref, bet_ref, o_ref):
    x = g_ref[...]
    seg = seg_ref[...]
    oneh = (seg == lax.broadcasted_iota(jnp.int32, (_N, 8), 1)
            ).astype(jnp.float32)
    t = lax.dot_general(oneh, typ_ref[...], (((1,), (0,)), ((), ())),
                        preferred_element_type=jnp.float32)
    x = x + pos_ref[...] + t
    ones = jnp.ones((DIM, 1), jnp.float32)
    dot = lambda a: lax.dot_general(a, ones, (((1,), (0,)), ((), ())),
                                    preferred_element_type=jnp.float32)
    ssum = dot(x)
    ssq = dot(x * x)
    mean = ssum * (1.0 / DIM)
    var = ssq * (1.0 / DIM) - mean * mean
    r = lax.rsqrt(var + EPS)
    o_ref[...] = (x - mean) * r * gam_ref[...] + bet_ref[...]


def _ln(gathered, segment_ids, pos16, type_table, gamma, beta):
    full = lambda shape: pl.BlockSpec(shape, lambda i: tuple(0 for _ in shape))
    typ8 = jnp.zeros((8, DIM), jnp.float32).at[:3].set(type_table)
    return pl.pallas_call(
        _ln_body,
        grid=(TOK // _N,),
        in_specs=[
            pl.BlockSpec((_N, DIM), lambda i: (i, 0)),
            pl.BlockSpec((_N, 1), lambda i: (i, 0)),
            full((_N, DIM)),
            full((8, DIM)),
            full((1, DIM)), full((1, DIM)),
        ],
        out_specs=pl.BlockSpec((_N, DIM), lambda i: (i, 0)),
        out_shape=jax.ShapeDtypeStruct((TOK, DIM), jnp.float32),
    )(gathered, segment_ids.reshape(TOK, 1),
      pos16, typ8,
      gamma.reshape(1, DIM), beta.reshape(1, DIM))


def kernel(input_ids, segment_ids, word_table, W2, pos_table, type_table,
           gamma, beta):
    table = _transform_table(word_table, W2)
    ids2d = input_ids.astype(jnp.int32).reshape(TOK // 128, 128)
    gathered = _sc_gather(table, ids2d)
    pos16 = jnp.tile(pos_table, (_BR, 1))
    out = _ln(gathered, segment_ids.astype(jnp.int32),
              pos16, type_table, gamma, beta)
    return out.reshape(B, MAX_LEN, DIM)


# speedup vs baseline: 1.6508x; 1.0000x over previous
import functools

import jax
import jax.numpy as jnp
from jax import lax
from jax.experimental import pallas as pl
from jax.experimental.pallas import tpu as pltpu
from jax.experimental.pallas import tpu_sc as plsc

VOCAB = 64001
DIM = 128
MAX_LEN = 200
B = 4096
TOK = B * MAX_LEN
EPS = 1e-12


def _transform_body(w_ref, w2_ref, o_ref):
    o_ref[...] = lax.dot_general(
        w_ref[...], w2_ref[...], (((1,), (1,)), ((), ())),
        preferred_element_type=jnp.float32)


def _transform_table(word_table, W2):
    R = 512
    return pl.pallas_call(
        _transform_body,
        grid=(pl.cdiv(VOCAB, R),),
        in_specs=[pl.BlockSpec((R, DIM), lambda i: (i, 0)),
                  pl.BlockSpec((DIM, DIM), lambda i: (0, 0))],
        out_specs=pl.BlockSpec((R, DIM), lambda i: (i, 0)),
        out_shape=jax.ShapeDtypeStruct((VOCAB, DIM), jnp.float32),
    )(word_table, W2)


_NW = 32
_B_PER_W = TOK // _NW
_CH = 256
_IR = _CH // 128
_NCH = _B_PER_W // _CH


def _sc_gather(table, ids2d):
    mesh = plsc.VectorSubcoreMesh(core_axis_name="c", subcore_axis_name="s")

    @functools.partial(
        pl.kernel,
        out_type=jax.ShapeDtypeStruct((TOK, DIM), jnp.float32),
        mesh=mesh,
        scratch_types=[
            pltpu.VMEM((2, _IR, 128), jnp.int32),
            pltpu.VMEM((2, _CH, DIM), jnp.float32),
            pltpu.SemaphoreType.DMA,
            pltpu.SemaphoreType.DMA,
            pltpu.SemaphoreType.DMA,
            pltpu.SemaphoreType.DMA,
        ],
        compiler_params=pltpu.CompilerParams(use_tc_tiling_on_sc=False),
    )
    def k(t_hbm, ids_hbm, out_hbm, idx_v, rows_v, sg0, sg1, so0, so1):
        wid = lax.axis_index("s") * 2 + lax.axis_index("c")
        sg = (sg0, sg1)
        so = (so0, so1)

        def issue(g, b):
            irow = wid * (_B_PER_W // 128) + g * _IR
            pltpu.sync_copy(ids_hbm.at[pl.ds(irow, _IR)], idx_v.at[b])
            for j in range(_IR):
                pltpu.async_copy(t_hbm.at[idx_v.at[b].at[j]],
                                 rows_v.at[b].at[pl.ds(j * 128, 128)], sg[b])

        def flush(g, b):
            for j in range(_IR):
                pltpu.make_async_copy(t_hbm.at[idx_v.at[b].at[j]],
                                      rows_v.at[b].at[pl.ds(j * 128, 128)],
                                      sg[b]).wait()
            base = wid * _B_PER_W + g * _CH
            pltpu.async_copy(rows_v.at[b], out_hbm.at[pl.ds(base, _CH)], so[b])

        def wait_out(b):
            pltpu.make_async_copy(rows_v.at[b], out_hbm.at[pl.ds(0, _CH)],
                                  so[b]).wait()

        issue(0, 0)

        def body(i, carry):
            g0 = 2 * i

            @pl.when(i >= 1)
            def _():
                wait_out(1)
            issue(g0 + 1, 1)
            flush(g0, 0)
            wait_out(0)

            @pl.when(g0 + 2 < _NCH)
            def _():
                issue(g0 + 2, 0)
            flush(g0 + 1, 1)
            return carry

        lax.fori_loop(0, _NCH // 2, body, 0)
        wait_out(1)

    return k(table, ids2d)


_BR = 16
_N = _BR * MAX_LEN


def _ln_body(g_ref, seg_ref, pos_ref, typ_ref, gam_ref, bet_ref, o_ref):
    x = g_ref[...]
    seg = seg_ref[...]
    oneh = (seg == lax.broadcasted_iota(jnp.int32, (_N, 8), 1)
            ).astype(jnp.float32)
    t = lax.dot_general(oneh, typ_ref[...], (((1,), (0,)), ((), ())),
                        preferred_element_type=jnp.float32)
    x = x + pos_ref[...] + t
    ones = jnp.ones((DIM, 1), jnp.float32)
    dot = lambda a: lax.dot_general(a, ones, (((1,), (0,)), ((), ())),
                                    preferred_element_type=jnp.float32)
    ssum = dot(x)
    ssq = dot(x * x)
    mean = ssum * (1.0 / DIM)
    var = ssq * (1.0 / DIM) - mean * mean
    r = lax.rsqrt(var + EPS)
    o_ref[...] = (x - mean) * r * gam_ref[...] + bet_ref[...]


def _ln(gathered, segment_ids, pos16, type_table, gamma, beta):
    full = lambda shape: pl.BlockSpec(shape, lambda i: tuple(0 for _ in shape))
    typ8 = jnp.zeros((8, DIM), jnp.float32).at[:3].set(type_table)
    return pl.pallas_call(
        _ln_body,
        grid=(TOK // _N,),
        in_specs=[
            pl.BlockSpec((_N, DIM), lambda i: (i, 0)),
            pl.BlockSpec((_N, 1), lambda i: (i, 0)),
            full((_N, DIM)),
            full((8, DIM)),
            full((1, DIM)), full((1, DIM)),
        ],
        out_specs=pl.BlockSpec((_N, DIM), lambda i: (i, 0)),
        out_shape=jax.ShapeDtypeStruct((TOK, DIM), jnp.float32),
    )(gathered, segment_ids.reshape(TOK, 1),
      pos16, typ8,
      gamma.reshape(1, DIM), beta.reshape(1, DIM))


def kernel(input_ids, segment_ids, word_table, W2, pos_table, type_table,
           gamma, beta):
    table = _transform_table(word_table, W2)
    ids2d = input_ids.astype(jnp.int32).reshape(TOK // 128, 128)
    gathered = _sc_gather(table, ids2d)
    pos16 = jnp.tile(pos_table, (_BR, 1))
    out = _ln(gathered, segment_ids.astype(jnp.int32),
              pos16, type_table, gamma, beta)
    return out.reshape(B, MAX_LEN, DIM)
